# baseline (speedup 1.0000x reference)
import functools

import jax
import jax.numpy as jnp
from jax import lax
from jax.experimental import pallas as pl
from jax.experimental.pallas import tpu as pltpu
from jax.experimental.pallas import tpu_sc as plsc

B = 16384
D = 64
OUT_D = 3 * D + 1 + D
PAD_D = 264
NC = 2
NS = 16
NW = NC * NS
RPW = B // NW
CHUNK = 128
NCHUNK = RPW // CHUNK
LANES = 16
NGRP = CHUNK // LANES

NBND = 120


def _bucketize(bnd_ref, ts16):
    pos = jnp.zeros((LANES,), jnp.int32)
    for step in (64, 32, 16, 8, 4, 2, 1):
        cand = pos + step
        safe = jnp.minimum(cand, NBND) - 1
        v = plsc.load_gather(bnd_ref, [safe])
        ok = (cand <= NBND) & (v < ts16)
        pos = jnp.where(ok, cand, pos)
    return pos


def _body(viz_hbm, ev_hbm, wt_hbm, ts_hbm, utab_hbm, atab_hbm, wtab_hbm,
          taug_hbm, tlast_hbm, bnd_hbm, tm_hbm, td_hbm, out_hbm,
          idx_v, ev_v, wt_v, ts_v, bk_v, ub0, ub1, ab0, ab1, wb0, wb1,
          tb0, tb1, tl0, tl1, bnd_v, tlast_v, tm_v, td_v,
          gsem0, gsem1, wsem0, wsem1):
    cid = lax.axis_index("c")
    sid = lax.axis_index("s")
    wid = sid * NC + cid
    base = wid * RPW

    pltpu.sync_copy(viz_hbm.at[pl.ds(base, RPW)], idx_v)
    pltpu.sync_copy(ev_hbm.at[pl.ds(base, RPW)], ev_v)
    pltpu.sync_copy(wt_hbm.at[pl.ds(base, RPW)], wt_v)
    pltpu.sync_copy(ts_hbm.at[pl.ds(base, RPW)], ts_v)
    pltpu.sync_copy(bnd_hbm, bnd_v)
    pltpu.sync_copy(tlast_hbm, tlast_v)
    pltpu.sync_copy(tm_hbm, tm_v)
    pltpu.sync_copy(td_hbm, td_v)

    tm16 = tm_v[...]
    td16 = td_v[...]
    lane = lax.iota(jnp.int32, LANES)
    zeros = jnp.zeros((LANES,), jnp.int32)
    bufs = ((ub0, ab0, wb0, tb0, tl0), (ub1, ab1, wb1, tb1, tl1))
    gsems = (gsem0, gsem1)
    wsems = (wsem0, wsem1)

    def fire_gathers(c):
        p = c % 2
        ub, ab, wb, tb, _ = bufs[p]
        sem = gsems[p]
        lo = c * CHUNK
        hs = [
            pltpu.async_copy(utab_hbm.at[idx_v.at[pl.ds(lo, CHUNK)]],
                             ub, sem),
            pltpu.async_copy(atab_hbm.at[ev_v.at[pl.ds(lo, CHUNK)]],
                             ab, sem),
            pltpu.async_copy(wtab_hbm.at[wt_v.at[pl.ds(lo, CHUNK)]],
                             wb, sem),
        ]
        for g in range(NGRP):
            ts16 = ts_v[pl.ds(lo + g * LANES, LANES)]
            bk16 = _bucketize(bnd_v, ts16)
            plsc.store_scatter(bk_v, [jnp.full((LANES,), p, jnp.int32),
                                      g * LANES + lane], bk16)
        hs.append(pltpu.async_copy(taug_hbm.at[bk_v.at[p]], tb, sem))
        return hs

    rows128 = pl.ds(0, CHUNK)
    ghandles = fire_gathers(0)
    whandles = None
    for c in range(NCHUNK):
        p = c % 2
        ub, ab, wb, tb, tl = bufs[p]
        for h in ghandles:
            h.wait()
        for g in range(NGRP):
            row16 = g * LANES + lane
            ts16 = ts_v[pl.ds(c * CHUNK + g * LANES, LANES)]
            cont = (ts16 - tm16) / td16
            plsc.store_scatter(tb, [row16, zeros], cont)
            bk16 = bk_v[p, pl.ds(g * LANES, LANES)]
            tl16 = plsc.load_gather(tlast_v, [bk16])
            plsc.store_scatter(tl, [row16, zeros], tl16)
        rows = pl.ds(base + c * CHUNK, CHUNK)
        wsem = wsems[p]
        whandles = [
            pltpu.async_copy(ub, out_hbm.at[rows, pl.ds(0, D)], wsem),
            pltpu.async_copy(ab, out_hbm.at[rows, pl.ds(D, D)], wsem),
            pltpu.async_copy(wb, out_hbm.at[rows, pl.ds(2 * D, D)], wsem),
            pltpu.async_copy(tb, out_hbm.at[rows, pl.ds(3 * D, D)], wsem),
            pltpu.async_copy(tl, out_hbm.at[rows, pl.ds(4 * D, 8)], wsem),
        ]
        if c + 1 < NCHUNK:
            if c >= 1:
                for h in prev_whandles:
                    h.wait()
            ghandles = fire_gathers(c + 1)
        prev_whandles = whandles
    for h in whandles:
        h.wait()


_sc_call = functools.partial(
    pl.kernel,
    out_type=jax.ShapeDtypeStruct((B, PAD_D), jnp.float32),
    mesh=plsc.VectorSubcoreMesh(core_axis_name="c", subcore_axis_name="s"),
    compiler_params=pltpu.CompilerParams(
        needs_layout_passes=False, use_tc_tiling_on_sc=False),
    scratch_types=[
        pltpu.VMEM((RPW,), jnp.int32),
        pltpu.VMEM((RPW,), jnp.int32),
        pltpu.VMEM((RPW,), jnp.int32),
        pltpu.VMEM((RPW,), jnp.float32),
        pltpu.VMEM((2, CHUNK), jnp.int32),
        pltpu.VMEM((CHUNK, D), jnp.float32),
        pltpu.VMEM((CHUNK, D), jnp.float32),
        pltpu.VMEM((CHUNK, D), jnp.float32),
        pltpu.VMEM((CHUNK, D), jnp.float32),
        pltpu.VMEM((CHUNK, D), jnp.float32),
        pltpu.VMEM((CHUNK, D), jnp.float32),
        pltpu.VMEM((CHUNK, D), jnp.float32),
        pltpu.VMEM((CHUNK, D), jnp.float32),
        pltpu.VMEM((CHUNK, 8), jnp.float32),
        pltpu.VMEM((CHUNK, 8), jnp.float32),
        pltpu.VMEM((128,), jnp.float32),
        pltpu.VMEM((128,), jnp.float32),
        pltpu.VMEM((LANES,), jnp.float32),
        pltpu.VMEM((LANES,), jnp.float32),
        pltpu.SemaphoreType.DMA,
        pltpu.SemaphoreType.DMA,
        pltpu.SemaphoreType.DMA,
        pltpu.SemaphoreType.DMA,
    ],
)(_body)


def kernel(visitorid, event, weight, timestamp, user_table, action_table,
           weight_table, time_table, boundaries, time_mean, time_std):
    viz = visitorid.astype(jnp.int32)
    ev = event.astype(jnp.int32)
    wt = weight.astype(jnp.int32)
    ts = timestamp.astype(jnp.float32)
    ttab = time_table.astype(jnp.float32)
    taug = jnp.concatenate(
        [jnp.zeros((ttab.shape[0], 1), jnp.float32), ttab[:, :D - 1]], axis=1)
    tlast = jnp.concatenate(
        [ttab[:, D - 1], jnp.zeros((128 - ttab.shape[0],), jnp.float32)])
    bnd = jnp.concatenate(
        [boundaries.astype(jnp.float32), jnp.zeros((8,), jnp.float32)])
    tm = jnp.full((LANES,), time_mean, jnp.float32)
    td = jnp.full((LANES,), time_std, jnp.float32)
    padded = _sc_call(viz, ev, wt, ts, user_table, action_table, weight_table,
                      taug, tlast, bnd, tm, td)
    return padded[:, :OUT_D]
